# baseline (speedup 1.0000x reference)
import functools

import jax
import jax.numpy as jnp
from jax import lax
from jax.experimental import pallas as pl
from jax.experimental.pallas import tpu as pltpu
from jax.experimental.pallas import tpu_sc as plsc

_EMBED = 64
_HIDDEN = 128
_NC, _NS = 2, 16
_NW = _NC * _NS
_C = 49152
_Q = _C // 4


def _fold_body(x_ref, i_ref, o_ref):
    xb = x_ref[...].astype(jnp.bfloat16)
    t = lax.dot_general(xb, i_ref[...], (((0,), (0,)), ((), ())),
                        preferred_element_type=jnp.float32)
    ti = lax.bitcast_convert_type(t, jnp.int32)
    a = ti[0 * _Q:1 * _Q]
    b = ti[1 * _Q:2 * _Q]
    c = ti[2 * _Q:3 * _Q]
    d = ti[3 * _Q:4 * _Q]
    hi_mask = jnp.int32(-65536)
    lo_ab = lax.shift_right_logical(a, 16) | (b & hi_mask)
    lo_cd = lax.shift_right_logical(c, 16) | (d & hi_mask)
    o = jnp.concatenate([lo_ab, lo_cd], axis=1)
    o_ref[...] = lax.bitcast_convert_type(o, jnp.float32)


def _tc_fold_pack(tbl_t):
    n = tbl_t.shape[1]
    nblk = pl.cdiv(n, _C)
    ident = jnp.eye(_EMBED, dtype=jnp.bfloat16)
    return pl.pallas_call(
        _fold_body,
        grid=(nblk,),
        in_specs=[
            pl.BlockSpec((_EMBED, _C), lambda k: (0, k)),
            pl.BlockSpec((_EMBED, _EMBED), lambda k: (0, 0)),
        ],
        out_specs=pl.BlockSpec((_Q, 2 * _EMBED), lambda k: (k, 0)),
        out_shape=jax.ShapeDtypeStruct((nblk * _Q, 2 * _EMBED), jnp.float32),
    )(tbl_t, ident)


def _sc_gather1(tbl2, idx):
    b = idx.shape[0]
    bw = b // _NW
    mesh = plsc.VectorSubcoreMesh(core_axis_name="c", subcore_axis_name="s")

    @functools.partial(
        pl.kernel,
        out_type=jax.ShapeDtypeStruct((b, 2 * _EMBED), jnp.float32),
        mesh=mesh,
        scratch_types=[
            pltpu.VMEM((bw,), jnp.int32),
            pltpu.VMEM((bw, 2 * _EMBED), jnp.float32),
            pltpu.SemaphoreType.DMA,
        ],
        compiler_params=pltpu.CompilerParams(use_tc_tiling_on_sc=True),
    )
    def k(t_hbm, i_hbm, o_hbm, i_v, r_v, sem):
        wid = lax.axis_index("s") * _NC + lax.axis_index("c")
        base = wid * bw
        half = bw // 2
        pltpu.sync_copy(i_hbm.at[pl.ds(base, bw)], i_v)
        c0 = pltpu.async_copy(
            t_hbm.at[i_v.at[pl.ds(0, half)]], r_v.at[pl.ds(0, half)], sem)
        c1 = pltpu.async_copy(
            t_hbm.at[i_v.at[pl.ds(half, half)]], r_v.at[pl.ds(half, half)],
            sem)
        c0.wait()
        pltpu.sync_copy(r_v.at[pl.ds(0, half)], o_hbm.at[pl.ds(base, half)])
        c1.wait()
        pltpu.sync_copy(r_v.at[pl.ds(half, half)],
                        o_hbm.at[pl.ds(base + half, half)])

    return k(tbl2, idx)


def _unpack_select(rows_ref, lo_ref, right_ref, lane_hi):
    ri = lax.bitcast_convert_type(rows_ref[...], jnp.int32)
    x_lo = lax.bitcast_convert_type(lax.shift_left(ri, 16), jnp.float32)
    x_hi = lax.bitcast_convert_type(ri & jnp.int32(-65536), jnp.float32)
    x = jnp.where(lo_ref[...] != 0, x_lo, x_hi)
    return jnp.where(lane_hi == (right_ref[...] != 0), x, 0.0)


def _mlp_body(u_ref, m_ref, ulo_ref, uhz_ref, mlo_ref, mhz_ref,
              w1u_ref, w1m_ref, b1_ref, w2_ref, b2_ref, o_ref):
    blk = u_ref.shape[0]
    lane = lax.broadcasted_iota(jnp.int32, (blk, 2 * _EMBED), 1)
    lane_hi = lane >= _EMBED
    u_x = _unpack_select(u_ref, ulo_ref, uhz_ref, lane_hi)
    m_x = _unpack_select(m_ref, mlo_ref, mhz_ref, lane_hi)
    h = (
        jnp.dot(u_x, w1u_ref[...], preferred_element_type=jnp.float32)
        + jnp.dot(m_x, w1m_ref[...], preferred_element_type=jnp.float32)
        + b1_ref[...]
    )
    h = jnp.maximum(h, 0.0)
    o_ref[...] = (
        jnp.dot(h, w2_ref[...], preferred_element_type=jnp.float32)
        + b2_ref[0, 0]
    )


def _tc_mlp(u_rows, m_rows, sel_u, sel_m, W1, b1, W2, b2):
    b = u_rows.shape[0]
    blk = 4096
    w1u_t = W1[:, :_EMBED].T
    w1m_t = W1[:, _EMBED:].T
    w1u2 = jnp.concatenate([w1u_t, w1u_t], axis=0)
    w1m2 = jnp.concatenate([w1m_t, w1m_t], axis=0)
    ulo, uhz = sel_u
    mlo, mhz = sel_m
    out = pl.pallas_call(
        _mlp_body,
        grid=(b // blk,),
        in_specs=[
            pl.BlockSpec((blk, 2 * _EMBED), lambda i: (i, 0)),
            pl.BlockSpec((blk, 2 * _EMBED), lambda i: (i, 0)),
            pl.BlockSpec((blk, 1), lambda i: (i, 0)),
            pl.BlockSpec((blk, 1), lambda i: (i, 0)),
            pl.BlockSpec((blk, 1), lambda i: (i, 0)),
            pl.BlockSpec((blk, 1), lambda i: (i, 0)),
            pl.BlockSpec((2 * _EMBED, _HIDDEN), lambda i: (0, 0)),
            pl.BlockSpec((2 * _EMBED, _HIDDEN), lambda i: (0, 0)),
            pl.BlockSpec((1, _HIDDEN), lambda i: (0, 0)),
            pl.BlockSpec((_HIDDEN, 1), lambda i: (0, 0)),
            pl.BlockSpec((1, 1), lambda i: (0, 0)),
        ],
        out_specs=pl.BlockSpec((blk, 1), lambda i: (i, 0)),
        out_shape=jax.ShapeDtypeStruct((b, 1), jnp.float32),
    )(u_rows, m_rows, ulo, uhz, mlo, mhz, w1u2, w1m2,
      b1.reshape(1, _HIDDEN), W2.reshape(_HIDDEN, 1), b2.reshape(1, 1))
    return out.reshape(b)


def _pack_index(i):
    blk = i // _C
    r = i % _C
    gi = blk * _Q + (r % _Q)
    q = r // _Q
    is_lo = 1 - (q & 1)
    is_right = q >> 1
    return gi, is_lo, is_right


def kernel(user_input, movie_input, user_table, movie_table, W1, b1, W2, b2):
    b = user_input.shape[0]
    ui = user_input.astype(jnp.int32)
    mi = movie_input.astype(jnp.int32)
    gu, ulo, uhz = _pack_index(ui)
    gm, mlo, mhz = _pack_index(mi)
    ut2 = _tc_fold_pack(user_table.T)
    u_rows = _sc_gather1(ut2, gu)
    mt2 = _tc_fold_pack(movie_table.T)
    m_rows = _sc_gather1(mt2, gm)
    r = lambda x: x.reshape(b, 1)
    return _tc_mlp(u_rows, m_rows, (r(ulo), r(uhz)), (r(mlo), r(mhz)),
                   W1, b1, W2, b2)


# speedup vs baseline: 1.0159x; 1.0159x over previous
import functools

import jax
import jax.numpy as jnp
from jax import lax
from jax.experimental import pallas as pl
from jax.experimental.pallas import tpu as pltpu
from jax.experimental.pallas import tpu_sc as plsc

_EMBED = 64
_HIDDEN = 128
_NC, _NS = 2, 16
_NW = _NC * _NS
_C = 49152
_Q = _C // 4


def _fold_body(x_ref, i_ref, o_ref):
    xb = x_ref[...].astype(jnp.bfloat16)
    t = lax.dot_general(xb, i_ref[...], (((0,), (0,)), ((), ())),
                        preferred_element_type=jnp.float32)
    ti = lax.bitcast_convert_type(t, jnp.int32)
    a = ti[0 * _Q:1 * _Q]
    b = ti[1 * _Q:2 * _Q]
    c = ti[2 * _Q:3 * _Q]
    d = ti[3 * _Q:4 * _Q]
    hi_mask = jnp.int32(-65536)
    lo_ab = lax.shift_right_logical(a, 16) | (b & hi_mask)
    lo_cd = lax.shift_right_logical(c, 16) | (d & hi_mask)
    o = jnp.concatenate([lo_ab, lo_cd], axis=1)
    o_ref[...] = lax.bitcast_convert_type(o, jnp.float32)


def _tc_fold_pack(tbl_t):
    n = tbl_t.shape[1]
    nblk = pl.cdiv(n, _C)
    ident = jnp.eye(_EMBED, dtype=jnp.bfloat16)
    return pl.pallas_call(
        _fold_body,
        grid=(nblk,),
        in_specs=[
            pl.BlockSpec((_EMBED, _C), lambda k: (0, k)),
            pl.BlockSpec((_EMBED, _EMBED), lambda k: (0, 0)),
        ],
        out_specs=pl.BlockSpec((_Q, 2 * _EMBED), lambda k: (k, 0)),
        out_shape=jax.ShapeDtypeStruct((nblk * _Q, 2 * _EMBED), jnp.float32),
    )(tbl_t, ident)


def _sc_gather1(tbl2, idx):
    b = idx.shape[0]
    bw = b // _NW
    mesh = plsc.VectorSubcoreMesh(core_axis_name="c", subcore_axis_name="s")

    @functools.partial(
        pl.kernel,
        out_type=jax.ShapeDtypeStruct((b, 2 * _EMBED), jnp.float32),
        mesh=mesh,
        scratch_types=[
            pltpu.VMEM((bw,), jnp.int32),
            pltpu.VMEM((bw, 2 * _EMBED), jnp.float32),
            pltpu.SemaphoreType.DMA,
        ],
        compiler_params=pltpu.CompilerParams(use_tc_tiling_on_sc=True),
    )
    def k(t_hbm, i_hbm, o_hbm, i_v, r_v, sem):
        wid = lax.axis_index("s") * _NC + lax.axis_index("c")
        base = wid * bw
        pltpu.sync_copy(i_hbm.at[pl.ds(base, bw)], i_v)
        pltpu.async_copy(t_hbm.at[i_v], r_v, sem).wait()
        pltpu.sync_copy(r_v, o_hbm.at[pl.ds(base, bw)])

    return k(tbl2, idx)


def _unpack_select(rows_ref, lo_ref, right_ref, lane_hi):
    ri = lax.bitcast_convert_type(rows_ref[...], jnp.int32)
    x_lo = lax.bitcast_convert_type(lax.shift_left(ri, 16), jnp.float32)
    x_hi = lax.bitcast_convert_type(ri & jnp.int32(-65536), jnp.float32)
    x = jnp.where(lo_ref[...] != 0, x_lo, x_hi)
    return jnp.where(lane_hi == (right_ref[...] != 0), x, 0.0)


def _mlp_body(u_ref, m_ref, ulo_ref, uhz_ref, mlo_ref, mhz_ref,
              w1u_ref, w1m_ref, b1_ref, w2_ref, b2_ref, o_ref):
    blk = u_ref.shape[0]
    lane = lax.broadcasted_iota(jnp.int32, (blk, 2 * _EMBED), 1)
    lane_hi = lane >= _EMBED
    u_x = _unpack_select(u_ref, ulo_ref, uhz_ref, lane_hi)
    m_x = _unpack_select(m_ref, mlo_ref, mhz_ref, lane_hi)
    h = (
        jnp.dot(u_x, w1u_ref[...], preferred_element_type=jnp.float32)
        + jnp.dot(m_x, w1m_ref[...], preferred_element_type=jnp.float32)
        + b1_ref[...]
    )
    h = jnp.maximum(h, 0.0)
    o_ref[...] = (
        jnp.dot(h, w2_ref[...], preferred_element_type=jnp.float32)
        + b2_ref[0, 0]
    )


def _tc_mlp(u_rows, m_rows, sel_u, sel_m, W1, b1, W2, b2):
    b = u_rows.shape[0]
    blk = 4096
    w1u_t = W1[:, :_EMBED].T
    w1m_t = W1[:, _EMBED:].T
    w1u2 = jnp.concatenate([w1u_t, w1u_t], axis=0)
    w1m2 = jnp.concatenate([w1m_t, w1m_t], axis=0)
    ulo, uhz = sel_u
    mlo, mhz = sel_m
    out = pl.pallas_call(
        _mlp_body,
        grid=(b // blk,),
        in_specs=[
            pl.BlockSpec((blk, 2 * _EMBED), lambda i: (i, 0)),
            pl.BlockSpec((blk, 2 * _EMBED), lambda i: (i, 0)),
            pl.BlockSpec((blk, 1), lambda i: (i, 0)),
            pl.BlockSpec((blk, 1), lambda i: (i, 0)),
            pl.BlockSpec((blk, 1), lambda i: (i, 0)),
            pl.BlockSpec((blk, 1), lambda i: (i, 0)),
            pl.BlockSpec((2 * _EMBED, _HIDDEN), lambda i: (0, 0)),
            pl.BlockSpec((2 * _EMBED, _HIDDEN), lambda i: (0, 0)),
            pl.BlockSpec((1, _HIDDEN), lambda i: (0, 0)),
            pl.BlockSpec((_HIDDEN, 1), lambda i: (0, 0)),
            pl.BlockSpec((1, 1), lambda i: (0, 0)),
        ],
        out_specs=pl.BlockSpec((blk, 1), lambda i: (i, 0)),
        out_shape=jax.ShapeDtypeStruct((b, 1), jnp.float32),
    )(u_rows, m_rows, ulo, uhz, mlo, mhz, w1u2, w1m2,
      b1.reshape(1, _HIDDEN), W2.reshape(_HIDDEN, 1), b2.reshape(1, 1))
    return out.reshape(b)


def _pack_index(i):
    blk = i // _C
    r = i % _C
    gi = blk * _Q + (r % _Q)
    q = r // _Q
    is_lo = 1 - (q & 1)
    is_right = q >> 1
    return gi, is_lo, is_right


def kernel(user_input, movie_input, user_table, movie_table, W1, b1, W2, b2):
    b = user_input.shape[0]
    ui = user_input.astype(jnp.int32)
    mi = movie_input.astype(jnp.int32)
    gu, ulo, uhz = _pack_index(ui)
    gm, mlo, mhz = _pack_index(mi)
    ut2 = _tc_fold_pack(user_table.T)
    u_rows = _sc_gather1(ut2, gu)
    mt2 = _tc_fold_pack(movie_table.T)
    m_rows = _sc_gather1(mt2, gm)
    r = lambda x: x.reshape(b, 1)
    return _tc_mlp(u_rows, m_rows, (r(ulo), r(uhz)), (r(mlo), r(mhz)),
                   W1, b1, W2, b2)
